# Initial kernel scaffold; baseline (speedup 1.0000x reference)
import functools

import jax
import jax.numpy as jnp
from jax.experimental import pallas as pl
from jax.experimental.pallas import tpu as pltpu

N = 10000
OUT = 64
CLS = 16
Hh = Ww = 384
HP = WP = 386


def _conv_lin_body(img_ref, w_ref, bias_ref, out_ref, buf, sem):
    def blk(i, _):
        cp = pltpu.make_async_copy(img_ref.at[pl.ds(i * 64, 66)], buf, sem)
        cp.start()
        cp.wait()

        def sub(j, _):
            rows = buf[pl.ds(j * 16, 18)]
            u = jnp.concatenate([rows[0:16], rows[1:17], rows[2:18]], axis=-1)
            c = jax.lax.dot_general(
                u.reshape(16 * 386, 192), w_ref[...],
                (((1,), (0,)), ((), ())),
                preferred_element_type=jnp.float32,
            ).reshape(16, 386, 48)
            o = (c[:, 0:384, 0:16] + c[:, 1:385, 16:32] + c[:, 2:386, 32:48]
                 + bias_ref[0:16])
            out_ref[pl.ds(i * 64 + j * 16, 16)] = o
            return 0

        jax.lax.fori_loop(0, 4, sub, 0)
        return 0

    jax.lax.fori_loop(0, 6, blk, 0)


def _conv_lin(img_pad, wcat, bias):
    return pl.pallas_call(
        _conv_lin_body,
        out_shape=jax.ShapeDtypeStruct((Hh, Ww, CLS), jnp.float32),
        in_specs=[
            pl.BlockSpec(memory_space=pltpu.ANY),
            pl.BlockSpec(memory_space=pltpu.VMEM),
            pl.BlockSpec(memory_space=pltpu.VMEM),
        ],
        out_specs=pl.BlockSpec(memory_space=pltpu.VMEM),
        scratch_shapes=[
            pltpu.VMEM((66, WP, OUT), jnp.float32),
            pltpu.SemaphoreType.DMA,
        ],
    )(img_pad, wcat, bias)


def kernel(x, edge_index, assign, W1, b1, W2, b2, conv_w, conv_b, lin_w, lin_b):
    h, w, c = x.shape
    x_flat = x.reshape(h * w, c)
    counts = jax.ops.segment_sum(jnp.ones((h * w,), jnp.float32), assign,
                                 num_segments=N)
    sp = jax.ops.segment_sum(x_flat, assign, num_segments=N) / \
        jnp.clip(counts, 1.0)[:, None]
    src, dst = edge_index[0], edge_index[1]

    def layer(X, W, b):
        half = W.shape[0] // 2
        Wa, Wb = W[:half], W[half:]
        P = X @ (Wa - Wb)
        Q = X @ Wb
        G = jax.ops.segment_max(Q[src], dst, num_segments=N)
        return jax.nn.relu(P + G + b)

    H1 = layer(sp, W1, b1)
    H2 = layer(H1, W2, b2)
    H3 = layer(H2, W2, b2)
    H4 = H1 + H2 + H3

    H4z = jnp.concatenate([H4, jnp.zeros((1, OUT), jnp.float32)], axis=0)
    yy, xx = jnp.meshgrid(jnp.arange(HP), jnp.arange(WP), indexing="ij")
    interior = (yy >= 1) & (yy < 385) & (xx >= 1) & (xx < 385)
    pix = (yy - 1) * Ww + (xx - 1)
    assign_pad = jnp.where(interior, assign[jnp.clip(pix, 0, Hh * Ww - 1)], N)
    img_pad = H4z[assign_pad.reshape(-1)].reshape(HP, WP, OUT)

    m = jnp.einsum("oikl,oc->klic", conv_w, lin_w)
    wcat = jnp.concatenate(
        [jnp.concatenate([m[ky, kx] for ky in range(3)], axis=0)
         for kx in range(3)], axis=1)
    bias16 = conv_b @ lin_w + lin_b
    bias = jnp.tile(bias16, 3)

    out = _conv_lin(img_pad, wcat, bias)
    return out.reshape(Hh * Ww, CLS)


# speedup vs baseline: 1.3972x; 1.3972x over previous
import functools

import jax
import jax.numpy as jnp
from jax.experimental import pallas as pl
from jax.experimental.pallas import tpu as pltpu

N = 10000
OUT = 64
CLS = 16
Hh = Ww = 384
HP = WP = 386


def _conv_lin_body(img_ref, w_ref, bias_ref, out_ref, buf, obuf, sem, osem):
    def blk(i, _):
        cp = pltpu.make_async_copy(img_ref.at[pl.ds(i * 64, 66)], buf, sem)
        cp.start()
        cp.wait()

        def sub(j, _):
            rows = buf[pl.ds(j * 16, 18)]
            u = jnp.concatenate([rows[0:16], rows[1:17], rows[2:18]], axis=-1)
            c = jax.lax.dot_general(
                u.reshape(16 * 386, 192), w_ref[...],
                (((1,), (0,)), ((), ())),
                preferred_element_type=jnp.float32,
            ).reshape(16, 386, 48)
            obuf[...] = (c[:, 0:384, 0:16] + c[:, 1:385, 16:32]
                         + c[:, 2:386, 32:48] + bias_ref[0:16])
            ocp = pltpu.make_async_copy(
                obuf, out_ref.at[pl.ds(i * 64 + j * 16, 16)], osem)
            ocp.start()
            ocp.wait()
            return 0

        jax.lax.fori_loop(0, 4, sub, 0)
        return 0

    jax.lax.fori_loop(0, 6, blk, 0)


def _conv_lin(img_pad, wcat, bias):
    return pl.pallas_call(
        _conv_lin_body,
        out_shape=jax.ShapeDtypeStruct((Hh, Ww, CLS), jnp.float32),
        in_specs=[
            pl.BlockSpec(memory_space=pl.ANY),
            pl.BlockSpec(memory_space=pltpu.VMEM),
            pl.BlockSpec(memory_space=pltpu.VMEM),
        ],
        out_specs=pl.BlockSpec(memory_space=pl.ANY),
        scratch_shapes=[
            pltpu.VMEM((66, WP, OUT), jnp.float32),
            pltpu.VMEM((16, Ww, CLS), jnp.float32),
            pltpu.SemaphoreType.DMA,
            pltpu.SemaphoreType.DMA,
        ],
    )(img_pad, wcat, bias)


def kernel(x, edge_index, assign, W1, b1, W2, b2, conv_w, conv_b, lin_w, lin_b):
    h, w, c = x.shape
    x_flat = x.reshape(h * w, c)
    counts = jax.ops.segment_sum(jnp.ones((h * w,), jnp.float32), assign,
                                 num_segments=N)
    sp = jax.ops.segment_sum(x_flat, assign, num_segments=N) / \
        jnp.clip(counts, 1.0)[:, None]
    src, dst = edge_index[0], edge_index[1]

    def layer(X, W, b):
        half = W.shape[0] // 2
        Wa, Wb = W[:half], W[half:]
        P = X @ (Wa - Wb)
        Q = X @ Wb
        G = jax.ops.segment_max(Q[src], dst, num_segments=N)
        return jax.nn.relu(P + G + b)

    H1 = layer(sp, W1, b1)
    H2 = layer(H1, W2, b2)
    H3 = layer(H2, W2, b2)
    H4 = H1 + H2 + H3

    H4z = jnp.concatenate([H4, jnp.zeros((1, OUT), jnp.float32)], axis=0)
    yy, xx = jnp.meshgrid(jnp.arange(HP), jnp.arange(WP), indexing="ij")
    interior = (yy >= 1) & (yy < 385) & (xx >= 1) & (xx < 385)
    pix = (yy - 1) * Ww + (xx - 1)
    assign_pad = jnp.where(interior, assign[jnp.clip(pix, 0, Hh * Ww - 1)], N)
    img_pad = H4z[assign_pad.reshape(-1)].reshape(HP, WP, OUT)

    m = jnp.einsum("oikl,oc->klic", conv_w, lin_w)
    wcat = jnp.concatenate(
        [jnp.concatenate([m[ky, kx] for ky in range(3)], axis=0)
         for kx in range(3)], axis=1)
    bias16 = conv_b @ lin_w + lin_b
    bias = jnp.tile(bias16, 3)

    out = _conv_lin(img_pad, wcat, bias)
    return out.reshape(Hh * Ww, CLS)
